# Initial kernel scaffold; baseline (speedup 1.0000x reference)
import jax
import jax.numpy as jnp
from jax.experimental import pallas as pl


def kernel(inputs, W0, W1, W2):
    raise NotImplementedError("write your pallas kernel here")


# speedup vs baseline: 1.4411x; 1.4411x over previous
import functools

import jax
import jax.numpy as jnp
from jax import lax
from jax.experimental import pallas as pl
from jax.experimental.pallas import tpu as pltpu
from jax.experimental.pallas import tpu_sc as plsc

_DIM = 768
_NC, _NS = 2, 16
_NW = _NC * _NS


@functools.partial(jax.jit, static_argnums=(1, 2))
def _gather3(idx_flat, B, b_per_w, W0, W1, W2):
    mesh = plsc.VectorSubcoreMesh(core_axis_name="c", subcore_axis_name="s")

    @functools.partial(
        pl.kernel,
        mesh=mesh,
        out_type=[jax.ShapeDtypeStruct((B, _DIM), jnp.float32)] * 3,
        scratch_types=[
            pltpu.VMEM((b_per_w,), jnp.int32),
            pltpu.VMEM((b_per_w, _DIM), jnp.float32),
            pltpu.SemaphoreType.DMA,
        ],
    )
    def k(idx_hbm, w0, w1, w2, o0, o1, o2, idx_v, rows_v, sem):
        wid = lax.axis_index("s") * _NC + lax.axis_index("c")
        base = wid * b_per_w
        pltpu.sync_copy(idx_hbm.at[pl.ds(base, b_per_w)], idx_v)
        for w, o in ((w0, o0), (w1, o1), (w2, o2)):
            pltpu.async_copy(w.at[idx_v], rows_v, sem).wait()
            pltpu.sync_copy(rows_v, o.at[pl.ds(base, b_per_w)])

    return k(idx_flat, W0, W1, W2)


def kernel(inputs, W0, W1, W2):
    Bc, S = inputs.shape
    B = Bc * S
    idx_flat = inputs.reshape(B)
    o0, o1, o2 = _gather3(idx_flat, B, B // _NW, W0, W1, W2)
    ve0 = o0.reshape(Bc, S, _DIM)
    ve1 = o1.reshape(Bc, S, _DIM)
    ve2 = o2.reshape(Bc, S, _DIM)
    return (ve0, ve1, ve2, ve0, ve1, ve2)


# speedup vs baseline: 1.4454x; 1.0030x over previous
import functools

import jax
import jax.numpy as jnp
from jax import lax
from jax.experimental import pallas as pl
from jax.experimental.pallas import tpu as pltpu
from jax.experimental.pallas import tpu_sc as plsc

_DIM = 768
_NC, _NS = 2, 16
_NW = _NC * _NS
_CH = 64


@functools.partial(jax.jit, static_argnums=(1, 2))
def _gather3(idx_3d, B, b_per_w, W0, W1, W2):
    n_ch = b_per_w // _CH
    mesh = plsc.VectorSubcoreMesh(core_axis_name="c", subcore_axis_name="s")

    @functools.partial(
        pl.kernel,
        mesh=mesh,
        out_type=[jax.ShapeDtypeStruct((B, _DIM), jnp.float32)] * 3,
        scratch_types=[
            pltpu.VMEM((n_ch, _CH), jnp.int32),
            pltpu.VMEM((_CH, _DIM), jnp.float32),
            pltpu.VMEM((_CH, _DIM), jnp.float32),
            pltpu.SemaphoreType.DMA,
            pltpu.SemaphoreType.DMA,
            pltpu.SemaphoreType.DMA,
            pltpu.SemaphoreType.DMA,
        ],
    )
    def k(idx_hbm, w0, w1, w2, o0, o1, o2,
          idx_v, buf_a, buf_b, gs0, gs1, ws0, ws1):
        wid = lax.axis_index("s") * _NC + lax.axis_index("c")
        base = wid * b_per_w
        bufs, gsems, wsems = (buf_a, buf_b), (gs0, gs1), (ws0, ws1)
        pltpu.sync_copy(idx_hbm.at[wid], idx_v)

        steps = [(w, o, c)
                 for (w, o) in ((w0, o0), (w1, o1), (w2, o2))
                 for c in range(n_ch)]
        n = len(steps)

        def start_gather(s):
            w, _, c = steps[s]
            b = s % 2
            return pltpu.async_copy(w.at[idx_v.at[c]], bufs[b], gsems[b])

        writes = [None] * n
        g = start_gather(0)
        for s in range(n):
            b = s % 2
            if s + 1 < n:
                if s >= 1:
                    writes[s - 1].wait()
                g_next = start_gather(s + 1)
            g.wait()
            _, o, c = steps[s]
            writes[s] = pltpu.async_copy(
                bufs[b], o.at[pl.ds(base + c * _CH, _CH)], wsems[b])
            if s + 1 < n:
                g = g_next
        writes[n - 2].wait()
        writes[n - 1].wait()

    return k(idx_3d, W0, W1, W2)


def kernel(inputs, W0, W1, W2):
    Bc, S = inputs.shape
    B = Bc * S
    b_per_w = B // _NW
    idx_3d = inputs.reshape(_NW, b_per_w // _CH, _CH)
    o0, o1, o2 = _gather3(idx_3d, B, b_per_w, W0, W1, W2)
    ve0 = o0.reshape(Bc, S, _DIM)
    ve1 = o1.reshape(Bc, S, _DIM)
    ve2 = o2.reshape(Bc, S, _DIM)
    return (ve0, ve1, ve2, ve0, ve1, ve2)


# speedup vs baseline: 1.8872x; 1.3056x over previous
import functools

import jax
import jax.numpy as jnp
from jax import lax
from jax.experimental import pallas as pl
from jax.experimental.pallas import tpu as pltpu
from jax.experimental.pallas import tpu_sc as plsc

_DIM = 768
_NC, _NS = 2, 16
_NW = _NC * _NS
_CH = 64


@functools.partial(jax.jit, static_argnums=(1, 2))
def _gather3(idx_3d, B, b_per_w, W0, W1, W2):
    n_ch = b_per_w // _CH
    mesh = plsc.VectorSubcoreMesh(core_axis_name="c", subcore_axis_name="s")

    @functools.partial(
        pl.kernel,
        mesh=mesh,
        out_type=[jax.ShapeDtypeStruct((B, _DIM), jnp.float32)] * 6,
        scratch_types=[
            pltpu.VMEM((n_ch, _CH), jnp.int32),
            pltpu.VMEM((_CH, _DIM), jnp.float32),
            pltpu.VMEM((_CH, _DIM), jnp.float32),
            pltpu.SemaphoreType.DMA,
            pltpu.SemaphoreType.DMA,
            pltpu.SemaphoreType.DMA,
            pltpu.SemaphoreType.DMA,
        ],
    )
    def k(idx_hbm, w0, w1, w2, o0, o1, o2, o3, o4, o5,
          idx_v, buf_a, buf_b, gs0, gs1, ws0, ws1):
        wid = lax.axis_index("s") * _NC + lax.axis_index("c")
        base = wid * b_per_w
        bufs, gsems, wsems = (buf_a, buf_b), (gs0, gs1), (ws0, ws1)
        pltpu.sync_copy(idx_hbm.at[wid], idx_v)

        steps = [(w, oa, ob, c)
                 for (w, oa, ob) in ((w0, o0, o3), (w1, o1, o4), (w2, o2, o5))
                 for c in range(n_ch)]
        n = len(steps)

        def start_gather(s):
            w, _, _, c = steps[s]
            b = s % 2
            return pltpu.async_copy(w.at[idx_v.at[c]], bufs[b], gsems[b])

        writes = [None] * n
        g = start_gather(0)
        for s in range(n):
            b = s % 2
            if s + 1 < n:
                if s >= 1:
                    for d in writes[s - 1]:
                        d.wait()
                g_next = start_gather(s + 1)
            g.wait()
            _, oa, ob, c = steps[s]
            dst = pl.ds(base + c * _CH, _CH)
            writes[s] = (
                pltpu.async_copy(bufs[b], oa.at[dst], wsems[b]),
                pltpu.async_copy(bufs[b], ob.at[dst], wsems[b]),
            )
            if s + 1 < n:
                g = g_next
        for s in (n - 2, n - 1):
            for d in writes[s]:
                d.wait()

    return k(idx_3d, W0, W1, W2)


def kernel(inputs, W0, W1, W2):
    Bc, S = inputs.shape
    B = Bc * S
    b_per_w = B // _NW
    idx_3d = inputs.reshape(_NW, b_per_w // _CH, _CH)
    outs = _gather3(idx_3d, B, b_per_w, W0, W1, W2)
    return tuple(o.reshape(Bc, S, _DIM) for o in outs)
